# Initial kernel scaffold; baseline (speedup 1.0000x reference)
import jax
import jax.numpy as jnp
from jax.experimental import pallas as pl


def kernel(x_, edge_index, W, bias):
    raise NotImplementedError("write your pallas kernel here")


# speedup vs baseline: 1.5311x; 1.5311x over previous
import jax
import jax.numpy as jnp
from jax import lax
from jax.experimental import pallas as pl
from jax.experimental.pallas import tpu as pltpu
from jax.experimental.pallas import tpu_sc as plsc

D = 128
E_EDGES = 320000

NC = 2
NS = 16
NW = NC * NS
NP = 10240
PT = NP // NW
CE = 6400
NCH = E_EDGES // CE
NCH2 = NCH // 2
GR = 128
NEG_INF = float("-inf")


def _matmul_body(x_ref, wat_ref, wjt_ref, b_ref, a_ref, bb_ref):
    xb = x_ref[...]
    a_ref[...] = (
        jnp.dot(xb, wat_ref[...], preferred_element_type=jnp.float32)
        + b_ref[...]
    )
    bb_ref[...] = jnp.dot(xb, wjt_ref[...], preferred_element_type=jnp.float32)


def _node_transform(xpad, wat, wjt, bias):
    blk = NP // 8
    return pl.pallas_call(
        _matmul_body,
        grid=(8,),
        in_specs=[
            pl.BlockSpec((blk, D), lambda i: (i, 0)),
            pl.BlockSpec((D, D), lambda i: (0, 0)),
            pl.BlockSpec((D, D), lambda i: (0, 0)),
            pl.BlockSpec((1, D), lambda i: (0, 0)),
        ],
        out_specs=[
            pl.BlockSpec((blk, D), lambda i: (i, 0)),
            pl.BlockSpec((blk, D), lambda i: (i, 0)),
        ],
        out_shape=[
            jax.ShapeDtypeStruct((NP, D), jnp.float32),
            jax.ShapeDtypeStruct((NP, D), jnp.float32),
        ],
    )(xpad, wat, wjt, bias)


def _seg_max_kernel(a_hbm, b_hbm, src_hbm, dst_hbm, out_hbm,
                    acc, sbuf0, dbuf0, sbuf1, dbuf1, csrc, cdst,
                    rows0, rows1, semc0, semc1, semg0, semg1):
    wid = lax.axis_index("s") * NC + lax.axis_index("c")
    lo = wid * PT

    def init_acc(i, _):
        acc[pl.ds(i * 16, 16)] = jnp.full((16,), NEG_INF, jnp.float32)
        return 0
    lax.fori_loop(0, (PT + 1) * D // 16, init_acc, 0)

    def init_comp(i, _):
        csrc[pl.ds(i * 16, 16)] = jnp.zeros((16,), jnp.int32)
        cdst[pl.ds(i * 16, 16)] = jnp.full((16,), PT, jnp.int32)
        return 0
    lax.fori_loop(0, (CE + 16) // 16, init_comp, 0)

    def issue_chunk(ch, sb, db, sem):
        pltpu.async_copy(src_hbm.at[pl.ds(ch * CE, CE)], sb, sem)
        pltpu.async_copy(dst_hbm.at[pl.ds(ch * CE, CE)], db, sem)

    def wait_chunk(sb, db, sem):
        pltpu.make_async_copy(src_hbm.at[pl.ds(0, CE)], sb, sem).wait()
        pltpu.make_async_copy(dst_hbm.at[pl.ds(0, CE)], db, sem).wait()

    def issue_g(gbuf, sem, g):
        idx = csrc.at[pl.ds(g * GR, GR)]
        pltpu.async_copy(b_hbm.at[idx], gbuf, sem)

    def wait_g(gbuf, sem):
        pltpu.make_async_copy(b_hbm.at[csrc.at[pl.ds(0, GR)]], gbuf, sem).wait()

    def process(sbuf, dbuf):
        def filt(i, cnt):
            for u in range(4):
                ii = i * 4 + u
                dv = dbuf[pl.ds(ii * 16, 16)]
                sv = sbuf[pl.ds(ii * 16, 16)]
                m = (dv >= lo) & (dv < lo + PT)
                mi = m.astype(jnp.int32)
                incl = plsc.cumsum(mi)
                idx = cnt + incl - mi
                plsc.store_scatter(csrc, [idx], sv, mask=m)
                plsc.store_scatter(cdst, [idx], dv - lo, mask=m)
                cnt = cnt + incl[15]
            return cnt
        cnt = lax.fori_loop(0, CE // 64, filt, jnp.int32(0))

        def accum_group(gbuf, g):
            start = g * GR
            nb = (jnp.minimum(cnt - start, GR) + 15) // 16

            def blk(q, _):
                dvec = cdst[pl.ds(start + q * 16, 16)]
                for j in range(16):
                    base = dvec[j] * D
                    eb = q * 16 + j
                    for c in range(D // 16):
                        sl = pl.ds(base + c * 16, 16)
                        acc[sl] = jnp.maximum(
                            acc[sl], gbuf[eb, pl.ds(c * 16, 16)]
                        )
                return 0
            lax.fori_loop(0, nb, blk, 0)

        ng = (cnt + GR - 1) // GR

        @pl.when(ng > 0)
        def _():
            issue_g(rows0, semg0, 0)

        def pair(p, _):
            g0 = 2 * p
            g1 = g0 + 1

            @pl.when(g1 < ng)
            def _():
                issue_g(rows1, semg1, g1)

            wait_g(rows0, semg0)
            accum_group(rows0, g0)

            @pl.when(g0 + 2 < ng)
            def _():
                issue_g(rows0, semg0, g0 + 2)

            @pl.when(g1 < ng)
            def _():
                wait_g(rows1, semg1)
                accum_group(rows1, g1)
            return 0
        lax.fori_loop(0, (ng + 1) // 2, pair, 0)

    issue_chunk(0, sbuf0, dbuf0, semc0)

    def chpair(cp, _):
        ch0 = 2 * cp
        issue_chunk(ch0 + 1, sbuf1, dbuf1, semc1)
        wait_chunk(sbuf0, dbuf0, semc0)
        process(sbuf0, dbuf0)

        @pl.when(cp < NCH2 - 1)
        def _():
            issue_chunk(ch0 + 2, sbuf0, dbuf0, semc0)

        wait_chunk(sbuf1, dbuf1, semc1)
        process(sbuf1, dbuf1)
        return 0
    lax.fori_loop(0, NCH2, chpair, 0)

    for b in range(PT // 64):
        pltpu.sync_copy(
            a_hbm.at[pl.ds(lo + b * 64, 64), :], rows0.at[pl.ds(0, 64), :]
        )

        def comb(r, _):
            off = (b * 64 + r) * D
            for c in range(D // 16):
                sl = pl.ds(off + c * 16, 16)
                acc[sl] = jnp.maximum(
                    acc[sl] + rows0[r, pl.ds(c * 16, 16)], 0.0
                )
            return 0
        lax.fori_loop(0, 64, comb, 0)

    pltpu.sync_copy(acc.at[pl.ds(0, PT * D)], out_hbm.at[pl.ds(lo * D, PT * D)])


def _segment_max_combine(a2d, b2d, src, dst):
    mesh = plsc.VectorSubcoreMesh(core_axis_name="c", subcore_axis_name="s")
    kfn = pl.kernel(
        _seg_max_kernel,
        mesh=mesh,
        out_type=jax.ShapeDtypeStruct((NP * D,), jnp.float32),
        scratch_types=[
            pltpu.VMEM(((PT + 1) * D,), jnp.float32),
            pltpu.VMEM((CE,), jnp.int32),
            pltpu.VMEM((CE,), jnp.int32),
            pltpu.VMEM((CE,), jnp.int32),
            pltpu.VMEM((CE,), jnp.int32),
            pltpu.VMEM((CE + 16,), jnp.int32),
            pltpu.VMEM((CE + 16,), jnp.int32),
            pltpu.VMEM((GR, D), jnp.float32),
            pltpu.VMEM((GR, D), jnp.float32),
            pltpu.SemaphoreType.DMA,
            pltpu.SemaphoreType.DMA,
            pltpu.SemaphoreType.DMA,
            pltpu.SemaphoreType.DMA,
        ],
        compiler_params=pltpu.CompilerParams(needs_layout_passes=False),
    )
    return kfn(a2d, b2d, src, dst)


def kernel(x_, edge_index, W, bias):
    b, HW, d = x_.shape
    x = x_.reshape(b * HW, d)
    xpad = jnp.pad(x, ((0, NP - b * HW), (0, 0)))

    wi = W[:, :d]
    wj = W[:, d:]
    wat = (wi - wj).T
    wjt = wj.T

    a2d, b2d = _node_transform(xpad, wat, wjt, bias.reshape(1, -1))

    out_flat = _segment_max_combine(a2d, b2d, edge_index[0], edge_index[1])
    out = out_flat.reshape(NP, D)[: b * HW]
    return out.reshape(b, HW, -1)


# speedup vs baseline: 2.2655x; 1.4796x over previous
import jax
import jax.numpy as jnp
from jax import lax
from jax.experimental import pallas as pl
from jax.experimental.pallas import tpu as pltpu
from jax.experimental.pallas import tpu_sc as plsc

D = 128
E_EDGES = 320000

NC = 2
NS = 16
NW = NC * NS
NP = 10240
CH = D // NW
CE = 6400
NCH = E_EDGES // CE
NCH2 = NCH // 2
NEG_INF = float("-inf")


def _matmul_body(xt_ref, wat_ref, wjt_ref, b_ref, a_ref, bb_ref):
    xb = xt_ref[...]
    a_ref[...] = (
        jnp.dot(wat_ref[...], xb, preferred_element_type=jnp.float32)
        + b_ref[...]
    )
    bb_ref[...] = jnp.dot(wjt_ref[...], xb, preferred_element_type=jnp.float32)


def _node_transform(xt, wa, wj, bias):
    blk = NP // 8
    return pl.pallas_call(
        _matmul_body,
        grid=(8,),
        in_specs=[
            pl.BlockSpec((D, blk), lambda i: (0, i)),
            pl.BlockSpec((D, D), lambda i: (0, 0)),
            pl.BlockSpec((D, D), lambda i: (0, 0)),
            pl.BlockSpec((D, 1), lambda i: (0, 0)),
        ],
        out_specs=[
            pl.BlockSpec((D, blk), lambda i: (0, i)),
            pl.BlockSpec((D, blk), lambda i: (0, i)),
        ],
        out_shape=[
            jax.ShapeDtypeStruct((D, NP), jnp.float32),
            jax.ShapeDtypeStruct((D, NP), jnp.float32),
        ],
    )(xt, wa, wj, bias)


def _seg_max_kernel(at_hbm, bt_hbm, src_hbm, dst_hbm, out_hbm,
                    acc, bsl, tmp, sbuf0, dbuf0, sbuf1, dbuf1,
                    semc0, semc1):
    wid = lax.axis_index("s") * NC + lax.axis_index("c")
    c_lo = wid * CH
    iota = lax.iota(jnp.int32, 16)
    ch4 = iota % CH

    pltpu.sync_copy(bt_hbm.at[pl.ds(c_lo, CH), :], bsl)

    def init_acc(i, _):
        for c in range(CH):
            acc[c, pl.ds(i * 16, 16)] = jnp.full((16,), NEG_INF, jnp.float32)
        return 0
    lax.fori_loop(0, NP // 16, init_acc, 0)

    def issue_chunk(ch, sb, db, sem):
        pltpu.async_copy(src_hbm.at[pl.ds(ch * CE, CE)], sb, sem)
        pltpu.async_copy(dst_hbm.at[pl.ds(ch * CE, CE)], db, sem)

    def wait_chunk(sb, db, sem):
        pltpu.make_async_copy(src_hbm.at[pl.ds(0, CE)], sb, sem).wait()
        pltpu.make_async_copy(dst_hbm.at[pl.ds(0, CE)], db, sem).wait()

    def process(sbuf, dbuf):
        def group(i, _):
            sv = sbuf[pl.ds(i * 16, 16)]
            dv = dbuf[pl.ds(i * 16, 16)]
            plsc.store_scatter(tmp, [dv], iota)
            rd = plsc.load_gather(tmp, [dv])
            ok = jnp.all(rd == iota)

            @pl.when(ok)
            def _():
                for c in range(CH):
                    cc = jnp.full((16,), c, jnp.int32)
                    bv = plsc.load_gather(bsl, [cc, sv])
                    av = plsc.load_gather(acc, [cc, dv])
                    plsc.store_scatter(acc, [cc, dv], jnp.maximum(av, bv))

            @pl.when(jnp.logical_not(ok))
            def _():
                for j in range(16):
                    sj = jnp.full((16,), sv[j], jnp.int32)
                    dj = jnp.full((16,), dv[j], jnp.int32)
                    bv = plsc.load_gather(bsl, [ch4, sj])
                    av = plsc.load_gather(acc, [ch4, dj])
                    plsc.store_scatter(acc, [ch4, dj], jnp.maximum(av, bv))
            return 0
        lax.fori_loop(0, CE // 16, group, 0)

    issue_chunk(0, sbuf0, dbuf0, semc0)

    def chpair(cp, _):
        ch0 = 2 * cp
        issue_chunk(ch0 + 1, sbuf1, dbuf1, semc1)
        wait_chunk(sbuf0, dbuf0, semc0)
        process(sbuf0, dbuf0)

        @pl.when(cp < NCH2 - 1)
        def _():
            issue_chunk(ch0 + 2, sbuf0, dbuf0, semc0)

        wait_chunk(sbuf1, dbuf1, semc1)
        process(sbuf1, dbuf1)
        return 0
    lax.fori_loop(0, NCH2, chpair, 0)

    pltpu.sync_copy(at_hbm.at[pl.ds(c_lo, CH), :], bsl)

    def comb(i, _):
        sl = pl.ds(i * 16, 16)
        for c in range(CH):
            acc[c, sl] = jnp.maximum(acc[c, sl] + bsl[c, sl], 0.0)
        return 0
    lax.fori_loop(0, NP // 16, comb, 0)

    pltpu.sync_copy(acc, out_hbm.at[pl.ds(c_lo, CH), :])


def _segment_max_combine(at2d, bt2d, src, dst):
    mesh = plsc.VectorSubcoreMesh(core_axis_name="c", subcore_axis_name="s")
    kfn = pl.kernel(
        _seg_max_kernel,
        mesh=mesh,
        out_type=jax.ShapeDtypeStruct((D, NP), jnp.float32),
        scratch_types=[
            pltpu.VMEM((CH, NP), jnp.float32),
            pltpu.VMEM((CH, NP), jnp.float32),
            pltpu.VMEM((NP,), jnp.int32),
            pltpu.VMEM((CE,), jnp.int32),
            pltpu.VMEM((CE,), jnp.int32),
            pltpu.VMEM((CE,), jnp.int32),
            pltpu.VMEM((CE,), jnp.int32),
            pltpu.SemaphoreType.DMA,
            pltpu.SemaphoreType.DMA,
        ],
        compiler_params=pltpu.CompilerParams(needs_layout_passes=False),
    )
    return kfn(at2d, bt2d, src, dst)


def kernel(x_, edge_index, W, bias):
    b, HW, d = x_.shape
    x = x_.reshape(b * HW, d)
    xpad = jnp.pad(x, ((0, NP - b * HW), (0, 0)))

    wi = W[:, :d]
    wj = W[:, d:]
    wa = wi - wj

    at2d, bt2d = _node_transform(xpad.T, wa, wj, bias.reshape(-1, 1))

    outt = _segment_max_combine(at2d, bt2d, edge_index[0], edge_index[1])
    out = outt.T[: b * HW]
    return out.reshape(b, HW, -1)


# speedup vs baseline: 2.3895x; 1.0547x over previous
import jax
import jax.numpy as jnp
from jax import lax
from jax.experimental import pallas as pl
from jax.experimental.pallas import tpu as pltpu
from jax.experimental.pallas import tpu_sc as plsc

D = 128
E_EDGES = 320000

NC = 2
NS = 16
NW = NC * NS
NP = 10240
CH = D // NW
CE = 6400
NCH = E_EDGES // CE
NCH2 = NCH // 2
NEG_INF = float("-inf")


def _matmul_body(xt_ref, wat_ref, wjt_ref, b_ref, a_ref, bb_ref):
    xb = xt_ref[...]
    a_ref[...] = (
        jnp.dot(wat_ref[...], xb, preferred_element_type=jnp.float32)
        + b_ref[...]
    )
    bb_ref[...] = jnp.dot(wjt_ref[...], xb, preferred_element_type=jnp.float32)


def _node_transform(xt, wa, wj, bias):
    blk = NP // 8
    return pl.pallas_call(
        _matmul_body,
        grid=(8,),
        in_specs=[
            pl.BlockSpec((D, blk), lambda i: (0, i)),
            pl.BlockSpec((D, D), lambda i: (0, 0)),
            pl.BlockSpec((D, D), lambda i: (0, 0)),
            pl.BlockSpec((D, 1), lambda i: (0, 0)),
        ],
        out_specs=[
            pl.BlockSpec((D, blk), lambda i: (0, i)),
            pl.BlockSpec((D, blk), lambda i: (0, i)),
        ],
        out_shape=[
            jax.ShapeDtypeStruct((D, NP), jnp.float32),
            jax.ShapeDtypeStruct((D, NP), jnp.float32),
        ],
    )(xt, wa, wj, bias)


def _seg_max_kernel(at_hbm, bt_hbm, src_hbm, dst_hbm, out_hbm,
                    acc, bsl, tmp, sbuf0, dbuf0, sbuf1, dbuf1,
                    semc0, semc1):
    wid = lax.axis_index("s") * NC + lax.axis_index("c")
    c_lo = wid * CH
    iota = lax.iota(jnp.int32, 16)
    ch4 = iota % CH

    pltpu.sync_copy(bt_hbm.at[pl.ds(c_lo, CH), :], bsl)

    def init_acc(i, _):
        for c in range(CH):
            acc[c, pl.ds(i * 16, 16)] = jnp.full((16,), NEG_INF, jnp.float32)
        return 0
    lax.fori_loop(0, NP // 16, init_acc, 0)

    def issue_chunk(ch, sb, db, sem):
        pltpu.async_copy(src_hbm.at[pl.ds(ch * CE, CE)], sb, sem)
        pltpu.async_copy(dst_hbm.at[pl.ds(ch * CE, CE)], db, sem)

    def wait_chunk(sb, db, sem):
        pltpu.make_async_copy(src_hbm.at[pl.ds(0, CE)], sb, sem).wait()
        pltpu.make_async_copy(dst_hbm.at[pl.ds(0, CE)], db, sem).wait()

    def process(sbuf, dbuf):
        ngrp = CE // 16

        def check(g):
            sv = sbuf[pl.ds(g * 16, 16)]
            dv = dbuf[pl.ds(g * 16, 16)]
            return sv, dv

        def group(i, carry):
            sv, dv, ok = carry
            nxt = jnp.minimum(i + 1, ngrp - 1)
            sv2, dv2 = check(nxt)
            plsc.store_scatter(tmp, [dv2], iota)

            for c in range(CH):
                cc = jnp.full((16,), c, jnp.int32)
                bv = plsc.load_gather(bsl, [cc, sv])
                av = plsc.load_gather(acc, [cc, dv])
                plsc.store_scatter(acc, [cc, dv], jnp.maximum(av, bv))

            rd2 = plsc.load_gather(tmp, [dv2])
            ok2 = jnp.all(rd2 == iota)

            @pl.when(jnp.logical_not(ok))
            def _():
                for j in range(16):
                    sj = jnp.full((16,), sv[j], jnp.int32)
                    dj = jnp.full((16,), dv[j], jnp.int32)
                    bv = plsc.load_gather(bsl, [ch4, sj])
                    av = plsc.load_gather(acc, [ch4, dj])
                    plsc.store_scatter(acc, [ch4, dj], jnp.maximum(av, bv))
            return sv2, dv2, ok2

        sv0, dv0 = check(0)
        plsc.store_scatter(tmp, [dv0], iota)
        rd0 = plsc.load_gather(tmp, [dv0])
        ok0 = jnp.all(rd0 == iota)
        lax.fori_loop(0, ngrp, group, (sv0, dv0, ok0))

    issue_chunk(0, sbuf0, dbuf0, semc0)

    def chpair(cp, _):
        ch0 = 2 * cp
        issue_chunk(ch0 + 1, sbuf1, dbuf1, semc1)
        wait_chunk(sbuf0, dbuf0, semc0)
        process(sbuf0, dbuf0)

        @pl.when(cp < NCH2 - 1)
        def _():
            issue_chunk(ch0 + 2, sbuf0, dbuf0, semc0)

        wait_chunk(sbuf1, dbuf1, semc1)
        process(sbuf1, dbuf1)
        return 0
    lax.fori_loop(0, NCH2, chpair, 0)

    pltpu.sync_copy(at_hbm.at[pl.ds(c_lo, CH), :], bsl)

    def comb(i, _):
        sl = pl.ds(i * 16, 16)
        for c in range(CH):
            acc[c, sl] = jnp.maximum(acc[c, sl] + bsl[c, sl], 0.0)
        return 0
    lax.fori_loop(0, NP // 16, comb, 0)

    pltpu.sync_copy(acc, out_hbm.at[pl.ds(c_lo, CH), :])


def _segment_max_combine(at2d, bt2d, src, dst):
    mesh = plsc.VectorSubcoreMesh(core_axis_name="c", subcore_axis_name="s")
    kfn = pl.kernel(
        _seg_max_kernel,
        mesh=mesh,
        out_type=jax.ShapeDtypeStruct((D, NP), jnp.float32),
        scratch_types=[
            pltpu.VMEM((CH, NP), jnp.float32),
            pltpu.VMEM((CH, NP), jnp.float32),
            pltpu.VMEM((NP,), jnp.int32),
            pltpu.VMEM((CE,), jnp.int32),
            pltpu.VMEM((CE,), jnp.int32),
            pltpu.VMEM((CE,), jnp.int32),
            pltpu.VMEM((CE,), jnp.int32),
            pltpu.SemaphoreType.DMA,
            pltpu.SemaphoreType.DMA,
        ],
        compiler_params=pltpu.CompilerParams(needs_layout_passes=False),
    )
    return kfn(at2d, bt2d, src, dst)


def kernel(x_, edge_index, W, bias):
    b, HW, d = x_.shape
    x = x_.reshape(b * HW, d)
    xpad = jnp.pad(x, ((0, NP - b * HW), (0, 0)))

    wi = W[:, :d]
    wj = W[:, d:]
    wa = wi - wj

    at2d, bt2d = _node_transform(xpad.T, wa, wj, bias.reshape(-1, 1))

    outt = _segment_max_combine(at2d, bt2d, edge_index[0], edge_index[1])
    out = outt.T[: b * HW]
    return out.reshape(b, HW, -1)


# speedup vs baseline: 3.6622x; 1.5327x over previous
import jax
import jax.numpy as jnp
from jax import lax
from jax.experimental import pallas as pl
from jax.experimental.pallas import tpu as pltpu
from jax.experimental.pallas import tpu_sc as plsc

D = 128
E_EDGES = 320000

NC = 2
NS = 16
NW = NC * NS
NP = 10240
CH = D // NW
CE = 6400
NCH = E_EDGES // CE
NCH2 = NCH // 2
NEG_INF = float("-inf")


def _matmul_body(xt_ref, wat_ref, wjt_ref, b_ref, a_ref, bb_ref):
    xb = xt_ref[...]
    a_ref[...] = (
        jnp.dot(wat_ref[...], xb, preferred_element_type=jnp.float32)
        + b_ref[...]
    )
    bb_ref[...] = jnp.dot(wjt_ref[...], xb, preferred_element_type=jnp.float32)


def _node_transform(xt, wa, wj, bias):
    blk = NP // 8
    return pl.pallas_call(
        _matmul_body,
        grid=(8,),
        in_specs=[
            pl.BlockSpec((D, blk), lambda i: (0, i)),
            pl.BlockSpec((D, D), lambda i: (0, 0)),
            pl.BlockSpec((D, D), lambda i: (0, 0)),
            pl.BlockSpec((D, 1), lambda i: (0, 0)),
        ],
        out_specs=[
            pl.BlockSpec((D, blk), lambda i: (0, i)),
            pl.BlockSpec((D, blk), lambda i: (0, i)),
        ],
        out_shape=[
            jax.ShapeDtypeStruct((D, NP), jnp.float32),
            jax.ShapeDtypeStruct((D, NP), jnp.float32),
        ],
    )(xt, wa, wj, bias)


def _seg_max_kernel(at_hbm, bt_hbm, src_hbm, dst_hbm, out_hbm,
                    acc, bsl, tmp, sbuf0, dbuf0, sbuf1, dbuf1,
                    semc0, semc1):
    wid = lax.axis_index("s") * NC + lax.axis_index("c")
    c_lo = wid * CH
    iota = lax.iota(jnp.int32, 16)
    ch4 = iota % CH

    pltpu.sync_copy(bt_hbm.at[pl.ds(c_lo, CH), :], bsl)

    def init_acc(i, _):
        for c in range(CH):
            acc[c, pl.ds(i * 16, 16)] = jnp.full((16,), NEG_INF, jnp.float32)
        return 0
    lax.fori_loop(0, NP // 16, init_acc, 0)

    def issue_chunk(ch, sb, db, sem):
        pltpu.async_copy(src_hbm.at[pl.ds(ch * CE, CE)], sb, sem)
        pltpu.async_copy(dst_hbm.at[pl.ds(ch * CE, CE)], db, sem)

    def wait_chunk(sb, db, sem):
        pltpu.make_async_copy(src_hbm.at[pl.ds(0, CE)], sb, sem).wait()
        pltpu.make_async_copy(dst_hbm.at[pl.ds(0, CE)], db, sem).wait()

    def process(sbuf, dbuf):
        ngrp = CE // 16

        def load_grp(g):
            sv = sbuf[pl.ds(g * 16, 16)]
            dv = dbuf[pl.ds(g * 16, 16)]
            return sv, dv

        def fast(sv, dv):
            bvs = [
                plsc.load_gather(bsl, [jnp.full((16,), c, jnp.int32), sv])
                for c in range(CH)
            ]
            avs = [
                plsc.load_gather(acc, [jnp.full((16,), c, jnp.int32), dv])
                for c in range(CH)
            ]
            for c in range(CH):
                plsc.store_scatter(
                    acc,
                    [jnp.full((16,), c, jnp.int32), dv],
                    jnp.maximum(avs[c], bvs[c]),
                )

        def fixup(sv, dv):
            for j in range(16):
                sj = jnp.full((16,), sv[j], jnp.int32)
                dj = jnp.full((16,), dv[j], jnp.int32)
                bv = plsc.load_gather(bsl, [ch4, sj])
                av = plsc.load_gather(acc, [ch4, dj])
                plsc.store_scatter(acc, [ch4, dj], jnp.maximum(av, bv))

        def check(g):
            sv, dv = load_grp(g)
            plsc.store_scatter(tmp, [dv], iota)
            rd = plsc.load_gather(tmp, [dv])
            return sv, dv, jnp.all(rd == iota)

        def group2(i, carry):
            sv_a, dv_a, ok_a = carry
            sv_b, dv_b, ok_b = check(2 * i + 1)
            fast(sv_a, dv_a)

            @pl.when(jnp.logical_not(ok_a))
            def _():
                fixup(sv_a, dv_a)

            nxt = jnp.minimum(2 * i + 2, ngrp - 1)
            sv_c, dv_c, ok_c = check(nxt)
            fast(sv_b, dv_b)

            @pl.when(jnp.logical_not(ok_b))
            def _():
                fixup(sv_b, dv_b)
            return sv_c, dv_c, ok_c

        carry0 = check(0)
        lax.fori_loop(0, ngrp // 2, group2, carry0)

    issue_chunk(0, sbuf0, dbuf0, semc0)

    def chpair(cp, _):
        ch0 = 2 * cp
        issue_chunk(ch0 + 1, sbuf1, dbuf1, semc1)
        wait_chunk(sbuf0, dbuf0, semc0)
        process(sbuf0, dbuf0)

        @pl.when(cp < NCH2 - 1)
        def _():
            issue_chunk(ch0 + 2, sbuf0, dbuf0, semc0)

        wait_chunk(sbuf1, dbuf1, semc1)
        process(sbuf1, dbuf1)
        return 0
    lax.fori_loop(0, NCH2, chpair, 0)

    pltpu.sync_copy(at_hbm.at[pl.ds(c_lo, CH), :], bsl)

    def comb(i, _):
        sl = pl.ds(i * 16, 16)
        for c in range(CH):
            acc[c, sl] = jnp.maximum(acc[c, sl] + bsl[c, sl], 0.0)
        return 0
    lax.fori_loop(0, NP // 16, comb, 0)

    pltpu.sync_copy(acc, out_hbm.at[pl.ds(c_lo, CH), :])


def _segment_max_combine(at2d, bt2d, src, dst):
    mesh = plsc.VectorSubcoreMesh(core_axis_name="c", subcore_axis_name="s")
    kfn = pl.kernel(
        _seg_max_kernel,
        mesh=mesh,
        out_type=jax.ShapeDtypeStruct((D, NP), jnp.float32),
        scratch_types=[
            pltpu.VMEM((CH, NP), jnp.float32),
            pltpu.VMEM((CH, NP), jnp.float32),
            pltpu.VMEM((NP,), jnp.int32),
            pltpu.VMEM((CE,), jnp.int32),
            pltpu.VMEM((CE,), jnp.int32),
            pltpu.VMEM((CE,), jnp.int32),
            pltpu.VMEM((CE,), jnp.int32),
            pltpu.SemaphoreType.DMA,
            pltpu.SemaphoreType.DMA,
        ],
        compiler_params=pltpu.CompilerParams(needs_layout_passes=False),
    )
    return kfn(at2d, bt2d, src, dst)


def kernel(x_, edge_index, W, bias):
    b, HW, d = x_.shape
    x = x_.reshape(b * HW, d)
    xpad = jnp.pad(x, ((0, NP - b * HW), (0, 0)))

    wi = W[:, :d]
    wj = W[:, d:]
    wa = wi - wj

    at2d, bt2d = _node_transform(xpad.T, wa, wj, bias.reshape(-1, 1))

    outt = _segment_max_combine(at2d, bt2d, edge_index[0], edge_index[1])
    out = outt.T[: b * HW]
    return out.reshape(b, HW, -1)


# speedup vs baseline: 3.8950x; 1.0636x over previous
import jax
import jax.numpy as jnp
from jax import lax
from jax.experimental import pallas as pl
from jax.experimental.pallas import tpu as pltpu
from jax.experimental.pallas import tpu_sc as plsc

D = 128
E_EDGES = 320000

NC = 2
NS = 16
NW = NC * NS
NP = 10240
CH = D // NW
CE = 6400
NCH = E_EDGES // CE
NCH2 = NCH // 2
NEG_INF = float("-inf")


def _matmul_body(xt_ref, wat_ref, wjt_ref, b_ref, a_ref, bb_ref):
    xb = xt_ref[...]
    a_ref[...] = (
        jnp.dot(wat_ref[...], xb, preferred_element_type=jnp.float32)
        + b_ref[...]
    )
    bb_ref[...] = jnp.dot(wjt_ref[...], xb, preferred_element_type=jnp.float32)


def _node_transform(xt, wa, wj, bias):
    blk = NP // 8
    return pl.pallas_call(
        _matmul_body,
        grid=(8,),
        in_specs=[
            pl.BlockSpec((D, blk), lambda i: (0, i)),
            pl.BlockSpec((D, D), lambda i: (0, 0)),
            pl.BlockSpec((D, D), lambda i: (0, 0)),
            pl.BlockSpec((D, 1), lambda i: (0, 0)),
        ],
        out_specs=[
            pl.BlockSpec((D, blk), lambda i: (0, i)),
            pl.BlockSpec((D, blk), lambda i: (0, i)),
        ],
        out_shape=[
            jax.ShapeDtypeStruct((D, NP), jnp.float32),
            jax.ShapeDtypeStruct((D, NP), jnp.float32),
        ],
    )(xt, wa, wj, bias)


def _seg_max_kernel(at_hbm, bt_hbm, src_hbm, dst_hbm, out_hbm,
                    acc, bsl, sbuf0, dbuf0, sbuf1, dbuf1,
                    semc0, semc1):
    wid = lax.axis_index("s") * NC + lax.axis_index("c")
    c_lo = wid * CH
    iota = lax.iota(jnp.int32, 16)
    ch4 = iota % CH

    pltpu.sync_copy(bt_hbm.at[pl.ds(c_lo, CH), :], bsl)

    def init_acc(i, _):
        for c in range(CH):
            acc[c, pl.ds(i * 16, 16)] = jnp.full((16,), NEG_INF, jnp.float32)
        return 0
    lax.fori_loop(0, NP // 16, init_acc, 0)

    def issue_chunk(ch, sb, db, sem):
        pltpu.async_copy(src_hbm.at[pl.ds(ch * CE, CE)], sb, sem)
        pltpu.async_copy(dst_hbm.at[pl.ds(ch * CE, CE)], db, sem)

    def wait_chunk(sb, db, sem):
        pltpu.make_async_copy(src_hbm.at[pl.ds(0, CE)], sb, sem).wait()
        pltpu.make_async_copy(dst_hbm.at[pl.ds(0, CE)], db, sem).wait()

    def process(sbuf, dbuf):
        ngrp = CE // 16

        def load_grp(g):
            sv = sbuf[pl.ds(g * 16, 16)]
            dv = dbuf[pl.ds(g * 16, 16)]
            return sv, dv

        def fast(sv, dv):
            bvs = [
                plsc.load_gather(bsl, [jnp.full((16,), c, jnp.int32), sv])
                for c in range(CH)
            ]
            avs = [
                plsc.load_gather(acc, [jnp.full((16,), c, jnp.int32), dv])
                for c in range(CH)
            ]
            for c in range(CH):
                plsc.store_scatter(
                    acc,
                    [jnp.full((16,), c, jnp.int32), dv],
                    jnp.maximum(avs[c], bvs[c]),
                )

        def fixup(sv, dv):
            for j in range(16):
                sj = jnp.full((16,), sv[j], jnp.int32)
                dj = jnp.full((16,), dv[j], jnp.int32)
                bv = plsc.load_gather(bsl, [ch4, sj])
                av = plsc.load_gather(acc, [ch4, dj])
                plsc.store_scatter(acc, [ch4, dj], jnp.maximum(av, bv))

        def check(g):
            sv, dv = load_grp(g)
            _, lastm = plsc.scan_count(dv)
            npop = plsc.all_reduce_population_count(lastm)
            return sv, dv, npop[0] == 16

        def group2(i, carry):
            sv_a, dv_a, ok_a = carry
            sv_b, dv_b, ok_b = check(2 * i + 1)
            fast(sv_a, dv_a)

            @pl.when(jnp.logical_not(ok_a))
            def _():
                fixup(sv_a, dv_a)

            nxt = jnp.minimum(2 * i + 2, ngrp - 1)
            sv_c, dv_c, ok_c = check(nxt)
            fast(sv_b, dv_b)

            @pl.when(jnp.logical_not(ok_b))
            def _():
                fixup(sv_b, dv_b)
            return sv_c, dv_c, ok_c

        carry0 = check(0)
        lax.fori_loop(0, ngrp // 2, group2, carry0)

    issue_chunk(0, sbuf0, dbuf0, semc0)

    def chpair(cp, _):
        ch0 = 2 * cp
        issue_chunk(ch0 + 1, sbuf1, dbuf1, semc1)
        wait_chunk(sbuf0, dbuf0, semc0)
        process(sbuf0, dbuf0)

        @pl.when(cp < NCH2 - 1)
        def _():
            issue_chunk(ch0 + 2, sbuf0, dbuf0, semc0)

        wait_chunk(sbuf1, dbuf1, semc1)
        process(sbuf1, dbuf1)
        return 0
    lax.fori_loop(0, NCH2, chpair, 0)

    pltpu.sync_copy(at_hbm.at[pl.ds(c_lo, CH), :], bsl)

    def comb(i, _):
        sl = pl.ds(i * 16, 16)
        for c in range(CH):
            acc[c, sl] = jnp.maximum(acc[c, sl] + bsl[c, sl], 0.0)
        return 0
    lax.fori_loop(0, NP // 16, comb, 0)

    pltpu.sync_copy(acc, out_hbm.at[pl.ds(c_lo, CH), :])


def _segment_max_combine(at2d, bt2d, src, dst):
    mesh = plsc.VectorSubcoreMesh(core_axis_name="c", subcore_axis_name="s")
    kfn = pl.kernel(
        _seg_max_kernel,
        mesh=mesh,
        out_type=jax.ShapeDtypeStruct((D, NP), jnp.float32),
        scratch_types=[
            pltpu.VMEM((CH, NP), jnp.float32),
            pltpu.VMEM((CH, NP), jnp.float32),
            pltpu.VMEM((CE,), jnp.int32),
            pltpu.VMEM((CE,), jnp.int32),
            pltpu.VMEM((CE,), jnp.int32),
            pltpu.VMEM((CE,), jnp.int32),
            pltpu.SemaphoreType.DMA,
            pltpu.SemaphoreType.DMA,
        ],
        compiler_params=pltpu.CompilerParams(needs_layout_passes=False),
    )
    return kfn(at2d, bt2d, src, dst)


def kernel(x_, edge_index, W, bias):
    b, HW, d = x_.shape
    x = x_.reshape(b * HW, d)
    xpad = jnp.pad(x, ((0, NP - b * HW), (0, 0)))

    wi = W[:, :d]
    wj = W[:, d:]
    wa = wi - wj

    at2d, bt2d = _node_transform(xpad.T, wa, wj, bias.reshape(-1, 1))

    outt = _segment_max_combine(at2d, bt2d, edge_index[0], edge_index[1])
    out = outt.T[: b * HW]
    return out.reshape(b, HW, -1)


# speedup vs baseline: 4.0025x; 1.0276x over previous
import jax
import jax.numpy as jnp
from jax import lax
from jax.experimental import pallas as pl
from jax.experimental.pallas import tpu as pltpu
from jax.experimental.pallas import tpu_sc as plsc

D = 128
E_EDGES = 320000

NC = 2
NS = 16
NW = NC * NS
NP = 10240
CH = D // NW
CE = 6400
NCH = E_EDGES // CE
NCH2 = NCH // 2
NEG_INF = float("-inf")


def _matmul_body(xt_ref, wat_ref, wjt_ref, b_ref, a_ref, bb_ref):
    xb = xt_ref[...]
    a_ref[...] = (
        jnp.dot(wat_ref[...], xb, preferred_element_type=jnp.float32)
        + b_ref[...]
    )
    bb_ref[...] = jnp.dot(wjt_ref[...], xb, preferred_element_type=jnp.float32)


def _node_transform(xt, wa, wj, bias):
    blk = NP // 8
    return pl.pallas_call(
        _matmul_body,
        grid=(8,),
        in_specs=[
            pl.BlockSpec((D, blk), lambda i: (0, i)),
            pl.BlockSpec((D, D), lambda i: (0, 0)),
            pl.BlockSpec((D, D), lambda i: (0, 0)),
            pl.BlockSpec((D, 1), lambda i: (0, 0)),
        ],
        out_specs=[
            pl.BlockSpec((D, blk), lambda i: (0, i)),
            pl.BlockSpec((D, blk), lambda i: (0, i)),
        ],
        out_shape=[
            jax.ShapeDtypeStruct((D, NP), jnp.float32),
            jax.ShapeDtypeStruct((D, NP), jnp.float32),
        ],
    )(xt, wa, wj, bias)


def _seg_max_kernel(at_hbm, bt_hbm, src_hbm, dst_hbm, out_hbm,
                    acc, bsl, sbuf0, dbuf0, sbuf1, dbuf1,
                    semc0, semc1):
    wid = lax.axis_index("s") * NC + lax.axis_index("c")
    c_lo = wid * CH
    iota = lax.iota(jnp.int32, 16)
    ch4 = iota % CH

    pltpu.sync_copy(bt_hbm.at[pl.ds(c_lo, CH), :], bsl)

    def init_acc(i, _):
        for c in range(CH):
            acc[c, pl.ds(i * 16, 16)] = jnp.full((16,), NEG_INF, jnp.float32)
        return 0
    lax.fori_loop(0, NP // 16, init_acc, 0)

    def issue_chunk(ch, sb, db, sem):
        pltpu.async_copy(src_hbm.at[pl.ds(ch * CE, CE)], sb, sem)
        pltpu.async_copy(dst_hbm.at[pl.ds(ch * CE, CE)], db, sem)

    def wait_chunk(sb, db, sem):
        pltpu.make_async_copy(src_hbm.at[pl.ds(0, CE)], sb, sem).wait()
        pltpu.make_async_copy(dst_hbm.at[pl.ds(0, CE)], db, sem).wait()

    def process(sbuf, dbuf):
        ngrp = CE // 16

        def load_grp(g):
            sv = sbuf[pl.ds(g * 16, 16)]
            dv = dbuf[pl.ds(g * 16, 16)]
            return sv, dv

        def fast(sv, dv):
            bvs = [
                plsc.load_gather(bsl, [jnp.full((16,), c, jnp.int32), sv])
                for c in range(CH)
            ]
            avs = [
                plsc.load_gather(acc, [jnp.full((16,), c, jnp.int32), dv])
                for c in range(CH)
            ]
            for c in range(CH):
                plsc.store_scatter(
                    acc,
                    [jnp.full((16,), c, jnp.int32), dv],
                    jnp.maximum(avs[c], bvs[c]),
                )

        def fixup(sv, dv):
            for j in range(16):
                sj = jnp.full((16,), sv[j], jnp.int32)
                dj = jnp.full((16,), dv[j], jnp.int32)
                bv = plsc.load_gather(bsl, [ch4, sj])
                av = plsc.load_gather(acc, [ch4, dj])
                plsc.store_scatter(acc, [ch4, dj], jnp.maximum(av, bv))

        def check(g):
            sv, dv = load_grp(g)
            _, lastm = plsc.scan_count(dv)
            npop = plsc.all_reduce_population_count(lastm)
            return sv, dv, npop[0] == 16

        UNROLL = 4

        def groupn(i, carry):
            for u in range(UNROLL):
                sv_a, dv_a, ok_a = carry
                nxt = UNROLL * i + u + 1
                if u == UNROLL - 1:
                    nxt = jnp.minimum(nxt, ngrp - 1)
                carry = check(nxt)
                fast(sv_a, dv_a)

                @pl.when(jnp.logical_not(ok_a))
                def _(sv_a=sv_a, dv_a=dv_a):
                    fixup(sv_a, dv_a)
            return carry

        carry0 = check(0)
        lax.fori_loop(0, ngrp // UNROLL, groupn, carry0)

    issue_chunk(0, sbuf0, dbuf0, semc0)

    def chpair(cp, _):
        ch0 = 2 * cp
        issue_chunk(ch0 + 1, sbuf1, dbuf1, semc1)
        wait_chunk(sbuf0, dbuf0, semc0)
        process(sbuf0, dbuf0)

        @pl.when(cp < NCH2 - 1)
        def _():
            issue_chunk(ch0 + 2, sbuf0, dbuf0, semc0)

        wait_chunk(sbuf1, dbuf1, semc1)
        process(sbuf1, dbuf1)
        return 0
    lax.fori_loop(0, NCH2, chpair, 0)

    pltpu.sync_copy(at_hbm.at[pl.ds(c_lo, CH), :], bsl)

    def comb(i, _):
        sl = pl.ds(i * 16, 16)
        for c in range(CH):
            acc[c, sl] = jnp.maximum(acc[c, sl] + bsl[c, sl], 0.0)
        return 0
    lax.fori_loop(0, NP // 16, comb, 0)

    pltpu.sync_copy(acc, out_hbm.at[pl.ds(c_lo, CH), :])


def _segment_max_combine(at2d, bt2d, src, dst):
    mesh = plsc.VectorSubcoreMesh(core_axis_name="c", subcore_axis_name="s")
    kfn = pl.kernel(
        _seg_max_kernel,
        mesh=mesh,
        out_type=jax.ShapeDtypeStruct((D, NP), jnp.float32),
        scratch_types=[
            pltpu.VMEM((CH, NP), jnp.float32),
            pltpu.VMEM((CH, NP), jnp.float32),
            pltpu.VMEM((CE,), jnp.int32),
            pltpu.VMEM((CE,), jnp.int32),
            pltpu.VMEM((CE,), jnp.int32),
            pltpu.VMEM((CE,), jnp.int32),
            pltpu.SemaphoreType.DMA,
            pltpu.SemaphoreType.DMA,
        ],
        compiler_params=pltpu.CompilerParams(needs_layout_passes=False),
    )
    return kfn(at2d, bt2d, src, dst)


def kernel(x_, edge_index, W, bias):
    b, HW, d = x_.shape
    x = x_.reshape(b * HW, d)
    xpad = jnp.pad(x, ((0, NP - b * HW), (0, 0)))

    wi = W[:, :d]
    wj = W[:, d:]
    wa = wi - wj

    at2d, bt2d = _node_transform(xpad.T, wa, wj, bias.reshape(-1, 1))

    outt = _segment_max_combine(at2d, bt2d, edge_index[0], edge_index[1])
    out = outt.T[: b * HW]
    return out.reshape(b, HW, -1)
